# Initial kernel scaffold; baseline (speedup 1.0000x reference)
import jax
import jax.numpy as jnp
from jax.experimental import pallas as pl


def kernel(h, e, edge_index, batch_idx, params):
    raise NotImplementedError("write your pallas kernel here")


# speedup vs baseline: 2.8911x; 2.8911x over previous
import functools

import jax
import jax.numpy as jnp
from jax import lax
from jax.experimental import pallas as pl
from jax.experimental.pallas import tpu as pltpu
from jax.experimental.pallas import tpu_sc as plsc

D = 128
H = 8
DH = 16
NBH = 128
EPS = 1e-5
R = 512
CH = 128
NSUB = 16
_SQRT2 = 1.4142135623730951


def _lnk(x, g, b):
    mu = jnp.mean(x, axis=-1, keepdims=True)
    xc = x - mu
    var = jnp.mean(xc * xc, axis=-1, keepdims=True)
    return xc * lax.rsqrt(var + EPS) * g + b


def _geluk(x):
    return 0.5 * x * (1.0 + lax.erf(x / _SQRT2))


def _sigk(x):
    return 1.0 / (1.0 + jnp.exp(-x))


def _gather_rows(h, idx3, epad):
    k_chunks = epad // (NSUB * CH)
    perw = k_chunks * CH
    mesh = plsc.VectorSubcoreMesh(core_axis_name="c", subcore_axis_name="s")

    @functools.partial(
        pl.kernel,
        mesh=mesh,
        out_type=(
            jax.ShapeDtypeStruct((epad, D), jnp.float32),
            jax.ShapeDtypeStruct((epad, D), jnp.float32),
        ),
        scratch_types=[
            pltpu.VMEM((k_chunks, CH), jnp.int32),
            pltpu.VMEM((CH, D), jnp.float32),
            pltpu.SemaphoreType.DMA,
        ],
    )
    def _gath(h_hbm, idx_hbm, hs_hbm, hd_hbm, idx_v, rows_v, sem):
        c = lax.axis_index("c")
        s = lax.axis_index("s")
        w = c * NSUB + s
        pltpu.sync_copy(idx_hbm.at[w], idx_v)
        base = s * perw

        def chunk(j, carry):
            pltpu.async_copy(h_hbm.at[idx_v.at[j]], rows_v, sem).wait()

            @pl.when(c == 0)
            def _():
                pltpu.sync_copy(rows_v, hs_hbm.at[pl.ds(base + j * CH, CH)])

            @pl.when(c == 1)
            def _():
                pltpu.sync_copy(rows_v, hd_hbm.at[pl.ds(base + j * CH, CH)])

            return carry

        lax.fori_loop(0, k_chunks, chunk, 0)

    return _gath(h, idx3)


def _tc1_body(n_edges, grid_n,
              hs_ref, hd_ref, e_ref, h_ref, b_ref,
              wqt, wkt, wvt, wot, we1at, we1bt, we1ct, we2t,
              wn1at, wn1bt, wn2t, v128, v256,
              hnew_ref, eout_ref, seg_ref, cnt_ref,
              seg_acc, cnt_acc):
    i = pl.program_id(0)
    f32 = jnp.float32
    hs = hs_ref[...]
    hd = hd_ref[...]
    ev = e_ref[...]
    hv = h_ref[...]

    bq = v128[0:1, :]
    bk = v128[1:2, :]
    bv = v128[2:3, :]
    bo = v128[3:4, :]
    be1 = v128[4:5, :]
    ge1 = v128[5:6, :]
    he1 = v128[6:7, :]
    be2 = v128[7:8, :]
    gl1 = v128[8:9, :]
    bl1 = v128[9:10, :]
    bn2 = v128[10:11, :]
    gl2 = v128[11:12, :]
    bl2 = v128[12:13, :]
    bn1 = v256[0:1, :]
    gn1 = v256[1:2, :]
    hn1 = v256[2:3, :]

    dot = functools.partial(jnp.dot, preferred_element_type=f32)

    q = dot(hd, wqt[...]) + bq
    k = dot(hs, wkt[...]) + bk
    v = dot(hs, wvt[...]) + bv
    row_h = lax.broadcasted_iota(jnp.int32, (D, D), 0) // DH
    col_h = lax.broadcasted_iota(jnp.int32, (D, D), 1) // DH
    smat = jnp.where(row_h == col_h, 0.25, 0.0).astype(f32)
    scores = dot(q * k, smat)
    m = jnp.max(scores, axis=-1, keepdims=True)
    p = jnp.exp(scores - m)
    denom = jnp.sum(p, axis=-1, keepdims=True) * (1.0 / DH)
    attn = p / denom
    msg = dot(attn * v, wot[...]) + bo

    e_mid = dot(hs, we1at[...]) + dot(ev, we1bt[...]) + dot(hd, we1ct[...]) + be1
    e_mid = _geluk(_lnk(e_mid, ge1, he1))
    e_upd = dot(e_mid, we2t[...]) + be2
    eout_ref[...] = _lnk(ev + e_upd, gl1, bl1)

    n_mid = dot(msg, wn1at[...]) + dot(hd, wn1bt[...]) + bn1
    n_mid = _geluk(_lnk(n_mid, gn1, hn1))
    h_upd = dot(n_mid, wn2t[...]) + bn2
    h_new = _lnk(hv + h_upd, gl2, bl2)
    hnew_ref[...] = h_new

    bi = b_ref[0]
    seg_ids = lax.broadcasted_iota(jnp.int32, (NBH, R), 0)
    pos = lax.broadcasted_iota(jnp.int32, (NBH, R), 1)
    valid = (pos + i * R) < n_edges
    oht = jnp.where((seg_ids == bi) & valid, 1.0, 0.0).astype(f32)

    @pl.when(i == 0)
    def _():
        seg_acc[...] = jnp.zeros_like(seg_acc)
        cnt_acc[...] = jnp.zeros_like(cnt_acc)

    seg_acc[...] += dot(oht, h_new)
    cnt_acc[...] = cnt_acc[...] + jnp.sum(oht, axis=-1, keepdims=True)

    @pl.when(i == grid_n - 1)
    def _():
        seg_ref[...] = seg_acc[...]
        cnt_ref[...] = cnt_acc[...]


def _tc2_body(hnew_ref, b_ref, seg, cnt, wg1t, wg2t, vg, out_ref, gates_s):
    i = pl.program_id(0)
    f32 = jnp.float32
    dot = functools.partial(jnp.dot, preferred_element_type=f32)

    @pl.when(i == 0)
    def _():
        bg1 = vg[0:1, :]
        gg1 = vg[1:2, :]
        hg1 = vg[2:3, :]
        bg2 = vg[3:4, :]
        h_global = seg[...] / jnp.maximum(cnt[...], 1.0)
        g_mid = dot(h_global, wg1t[...]) + bg1
        g_mid = _geluk(_lnk(g_mid, gg1, hg1))
        gates_s[...] = _sigk(dot(g_mid, wg2t[...]) + bg2)

    bi = b_ref[0]
    seg_ids = lax.broadcasted_iota(jnp.int32, (NBH, R), 0)
    oht = jnp.where(seg_ids == bi, 1.0, 0.0).astype(f32)
    gate_rows = lax.dot_general(oht, gates_s[...], (((0,), (0,)), ((), ())),
                                preferred_element_type=f32)
    out_ref[...] = hnew_ref[...] * gate_rows


def kernel(h, e, edge_index, batch_idx, params):
    n_nodes, d = h.shape
    n_edges = edge_index.shape[1]
    epad = ((n_edges + NSUB * CH - 1) // (NSUB * CH)) * (NSUB * CH)
    grid_n = (epad + R - 1) // R
    p = params

    src = edge_index[0].astype(jnp.int32)
    dst = edge_index[1].astype(jnp.int32)
    pad = jnp.zeros((epad - n_edges,), jnp.int32)
    k_chunks = epad // (NSUB * CH)
    idx3 = jnp.concatenate([src, pad, dst, pad]).reshape(2 * NSUB, k_chunks, CH)

    hs, hd = _gather_rows(h, idx3, epad)

    bpad = jnp.full((epad - n_edges,), NBH + 7, jnp.int32)
    batch3 = jnp.concatenate([batch_idx.astype(jnp.int32), bpad]).reshape(
        grid_n, 1, R)

    wqt = p["Wq"].T
    wkt = p["Wk"].T
    wvt = p["Wv"].T
    wot = p["Wo"].T
    we1t = p["We1"].T
    we1at = we1t[0:D]
    we1bt = we1t[D:2 * D]
    we1ct = we1t[2 * D:3 * D]
    we2t = p["We2"].T
    wn1t = p["Wn1"].T
    wn1at = wn1t[0:D]
    wn1bt = wn1t[D:2 * D]
    wn2t = p["Wn2"].T
    wg1t = p["Wg1"].T
    wg2t = p["Wg2"].T

    z128 = jnp.zeros((1, D), jnp.float32)
    v128 = jnp.concatenate([
        p["bq"][None], p["bk"][None], p["bv"][None], p["bo"][None],
        p["be1"][None], p["ge1"][None], p["he1"][None], p["be2"][None],
        p["g_ln1"][None], p["b_ln1"][None], p["bn2"][None],
        p["g_ln2"][None], p["b_ln2"][None], z128, z128, z128], axis=0)
    z256 = jnp.zeros((1, 2 * D), jnp.float32)
    v256 = jnp.concatenate([
        p["bn1"][None], p["gn1"][None], p["hn1"][None],
        z256, z256, z256, z256, z256], axis=0)
    vg = jnp.concatenate([
        p["bg1"][None], p["gg1"][None], p["hg1"][None], p["bg2"][None],
        z128, z128, z128, z128], axis=0)

    full = lambda shape: pl.BlockSpec(shape, lambda i: (0,) * len(shape))
    rowblk = pl.BlockSpec((R, D), lambda i: (i, 0))

    h_new, e_out, seg, cnt = pl.pallas_call(
        functools.partial(_tc1_body, n_edges, grid_n),
        grid=(grid_n,),
        in_specs=[
            rowblk, rowblk, rowblk, rowblk,
            pl.BlockSpec((1, 1, R), lambda i: (i, 0, 0)),
            full((D, D)), full((D, D)), full((D, D)), full((D, D)),
            full((D, D)), full((D, D)), full((D, D)), full((D, D)),
            full((D, 2 * D)), full((D, 2 * D)), full((2 * D, D)),
            full((16, D)), full((8, 2 * D)),
        ],
        out_specs=[
            rowblk, rowblk,
            full((NBH, D)), full((NBH, D)),
        ],
        out_shape=[
            jax.ShapeDtypeStruct((n_edges, D), jnp.float32),
            jax.ShapeDtypeStruct((n_edges, D), jnp.float32),
            jax.ShapeDtypeStruct((NBH, D), jnp.float32),
            jax.ShapeDtypeStruct((NBH, D), jnp.float32),
        ],
        scratch_shapes=[
            pltpu.VMEM((NBH, D), jnp.float32),
            pltpu.VMEM((NBH, D), jnp.float32),
        ],
        compiler_params=pltpu.CompilerParams(
            dimension_semantics=("arbitrary",)),
    )(hs, hd, e, h, batch3,
      wqt, wkt, wvt, wot, we1at, we1bt, we1ct, we2t,
      wn1at, wn1bt, wn2t, v128, v256)

    h_final = pl.pallas_call(
        _tc2_body,
        grid=(grid_n,),
        in_specs=[
            rowblk,
            pl.BlockSpec((1, 1, R), lambda i: (i, 0, 0)),
            full((NBH, D)), full((NBH, D)),
            full((D, D)), full((D, D)), full((8, D)),
        ],
        out_specs=rowblk,
        out_shape=jax.ShapeDtypeStruct((n_nodes, D), jnp.float32),
        scratch_shapes=[pltpu.VMEM((NBH, D), jnp.float32)],
        compiler_params=pltpu.CompilerParams(
            dimension_semantics=("arbitrary",)),
    )(h_new, batch3, seg, cnt, wg1t, wg2t, vg)

    return (h_final, e_out)
